# Initial kernel scaffold; baseline (speedup 1.0000x reference)
import jax
import jax.numpy as jnp
from jax.experimental import pallas as pl


def kernel(x, edge_index, W1_l, b1_l, W1_r, W2_l, b2_l, W2_r):
    raise NotImplementedError("write your pallas kernel here")


# speedup vs baseline: 5.2384x; 5.2384x over previous
import functools

import jax
import jax.numpy as jnp
from jax import lax
from jax.experimental import pallas as pl
from jax.experimental.pallas import tpu as pltpu
from jax.experimental.pallas import tpu_sc as plsc

N = 10000
E = 320000
D = 128

NC = 2
NS = 16
NW = NC * NS
EPW = E // NW
BATCH = 80
NITER = EPW // BATCH
NPAD = 10240
RPT = NPAD // NS
CH = 128
NCH = RPT // CH

_mesh = plsc.VectorSubcoreMesh(core_axis_name="c", subcore_axis_name="s")


@functools.partial(
    pl.kernel,
    mesh=_mesh,
    compiler_params=pltpu.CompilerParams(use_tc_tiling_on_sc=False),
    out_type=[
        jax.ShapeDtypeStruct((NC, NPAD, D), jnp.float32),
        jax.ShapeDtypeStruct((NC, NPAD, 16), jnp.float32),
    ],
    scratch_types=[
        pltpu.VMEM((BATCH,), jnp.int32),
        pltpu.VMEM((BATCH,), jnp.int32),
        pltpu.VMEM((BATCH, D), jnp.float32),
        pltpu.VMEM((BATCH, 16), jnp.float32),
        pltpu.VMEM((CH, D), jnp.float32),
        pltpu.VMEM((CH, 16), jnp.float32),
        pltpu.VMEM_SHARED((NPAD, D), jnp.float32),
        pltpu.VMEM_SHARED((NPAD, 16), jnp.float32),
        pltpu.SemaphoreType.DMA,
    ],
)
def _sc_agg(t_hbm, src_hbm, dst_hbm, zrow_hbm, zcnt_hbm, ones_hbm,
            pacc_hbm, pcnt_hbm,
            src_v, dst_v, rows_v, ones_v, stage_v, stagec_v, acc_s, cnt_s, sem):
    c = lax.axis_index("c")
    s = lax.axis_index("s")
    wid = s * NC + c

    pltpu.sync_copy(zrow_hbm, stage_v)
    pltpu.sync_copy(zcnt_hbm, stagec_v)
    pltpu.sync_copy(ones_hbm, ones_v)

    def zbody(j, carry):
        off = pl.multiple_of(s * RPT + j * CH, 8)
        pltpu.sync_copy(stage_v, acc_s.at[pl.ds(off, CH)])
        pltpu.sync_copy(stagec_v, cnt_s.at[pl.ds(off, CH)])
        return carry

    lax.fori_loop(0, NCH, zbody, 0)
    plsc.subcore_barrier()

    def body(i, carry):
        base = pl.multiple_of(wid * EPW + i * BATCH, 8)
        pltpu.sync_copy(src_hbm.at[pl.ds(base, BATCH)], src_v)
        pltpu.sync_copy(dst_hbm.at[pl.ds(base, BATCH)], dst_v)
        pltpu.async_copy(t_hbm.at[src_v], rows_v, sem).wait()
        pltpu.sync_copy(rows_v, acc_s.at[dst_v], add=True)
        pltpu.sync_copy(ones_v, cnt_s.at[dst_v], add=True)
        return carry

    lax.fori_loop(0, NITER, body, 0)
    plsc.subcore_barrier()

    def wbody(j, carry):
        off = pl.multiple_of(s * RPT + j * CH, 8)
        pltpu.sync_copy(acc_s.at[pl.ds(off, CH)], stage_v)
        pltpu.sync_copy(stage_v, pacc_hbm.at[c, pl.ds(off, CH)])
        pltpu.sync_copy(cnt_s.at[pl.ds(off, CH)], stagec_v)
        pltpu.sync_copy(stagec_v, pcnt_hbm.at[c, pl.ds(off, CH)])
        return carry

    lax.fori_loop(0, NCH, wbody, 0)


_BLK = 1000
_GRID = N // _BLK


def _dot_t(a, w):
    return lax.dot_general(a, w, (((1,), (1,)), ((), ())),
                           preferred_element_type=jnp.float32)


def _tc_in_body(x_ref, wl_ref, wr_ref, b_ref, t_ref, r_ref):
    xb = x_ref[...]
    t_ref[...] = _dot_t(xb, wl_ref[...])
    r_ref[...] = _dot_t(xb, wr_ref[...]) + b_ref[...]


def _tc_in(x, W_l, W_r, b):
    w_spec = pl.BlockSpec((D, D), lambda i: (0, 0))
    return pl.pallas_call(
        _tc_in_body,
        grid=(_GRID,),
        in_specs=[pl.BlockSpec((_BLK, D), lambda i: (i, 0)), w_spec, w_spec,
                  pl.BlockSpec((1, D), lambda i: (0, 0))],
        out_specs=[pl.BlockSpec((_BLK, D), lambda i: (i, 0))] * 2,
        out_shape=[jax.ShapeDtypeStruct((N, D), jnp.float32)] * 2,
    )(x, W_l, W_r, b.reshape(1, D))


def _tc_mid_body(pacc_ref, pcnt_ref, r_ref, wl_ref, wr_ref, b_ref,
                 t_ref, r2_ref):
    agg = pacc_ref[0] + pacc_ref[1]
    cnt = pcnt_ref[0, :, 0:1] + pcnt_ref[1, :, 0:1]
    mean = agg / jnp.maximum(cnt, 1.0)
    h = jnp.maximum(mean + r_ref[...], 0.0)
    t_ref[...] = _dot_t(h, wl_ref[...])
    r2_ref[...] = _dot_t(h, wr_ref[...]) + b_ref[...]


def _tc_mid(pacc, pcnt, r1, W_l, W_r, b):
    w_spec = pl.BlockSpec((D, D), lambda i: (0, 0))
    return pl.pallas_call(
        _tc_mid_body,
        grid=(_GRID,),
        in_specs=[pl.BlockSpec((NC, _BLK, D), lambda i: (0, i, 0)),
                  pl.BlockSpec((NC, _BLK, 16), lambda i: (0, i, 0)),
                  pl.BlockSpec((_BLK, D), lambda i: (i, 0)),
                  w_spec, w_spec,
                  pl.BlockSpec((1, D), lambda i: (0, 0))],
        out_specs=[pl.BlockSpec((_BLK, D), lambda i: (i, 0))] * 2,
        out_shape=[jax.ShapeDtypeStruct((N, D), jnp.float32)] * 2,
    )(pacc, pcnt, r1, W_l, W_r, b.reshape(1, D))


def _tc_out_body(pacc_ref, pcnt_ref, r_ref, o_ref):
    agg = pacc_ref[0] + pacc_ref[1]
    cnt = pcnt_ref[0, :, 0:1] + pcnt_ref[1, :, 0:1]
    o_ref[...] = agg / jnp.maximum(cnt, 1.0) + r_ref[...]


def _tc_out(pacc, pcnt, r2):
    return pl.pallas_call(
        _tc_out_body,
        grid=(_GRID,),
        in_specs=[pl.BlockSpec((NC, _BLK, D), lambda i: (0, i, 0)),
                  pl.BlockSpec((NC, _BLK, 16), lambda i: (0, i, 0)),
                  pl.BlockSpec((_BLK, D), lambda i: (i, 0))],
        out_specs=pl.BlockSpec((_BLK, D), lambda i: (i, 0)),
        out_shape=jax.ShapeDtypeStruct((N, D), jnp.float32),
    )(pacc, pcnt, r2)


def kernel(x, edge_index, W1_l, b1_l, W1_r, W2_l, b2_l, W2_r):
    src = edge_index[0].astype(jnp.int32)
    dst = edge_index[1].astype(jnp.int32)
    zrow = jnp.zeros((CH, D), jnp.float32)
    zcnt = jnp.zeros((CH, 16), jnp.float32)
    ones = jnp.ones((BATCH, 16), jnp.float32)

    t1, r1 = _tc_in(x, W1_l, W1_r, b1_l)
    pacc1, pcnt1 = _sc_agg(t1, src, dst, zrow, zcnt, ones)
    t2, r2 = _tc_mid(pacc1, pcnt1, r1, W2_l, W2_r, b2_l)
    pacc2, _ = _sc_agg(t2, src, dst, zrow, zcnt, ones)
    return _tc_out(pacc2, pcnt1, r2)


# speedup vs baseline: 9.5148x; 1.8163x over previous
import functools

import jax
import jax.numpy as jnp
from jax import lax
from jax.experimental import pallas as pl
from jax.experimental.pallas import tpu as pltpu
from jax.experimental.pallas import tpu_sc as plsc

N = 10000
E = 320000
D = 128

NC = 2
NS = 16
NW = NC * NS
EPW = E // NW
BATCH = 80
NITER = EPW // BATCH
NPAD = 10240
RPT = NPAD // NS
CH = 64
NCH = RPT // CH

_mesh = plsc.VectorSubcoreMesh(core_axis_name="c", subcore_axis_name="s")


@functools.partial(
    pl.kernel,
    mesh=_mesh,
    compiler_params=pltpu.CompilerParams(use_tc_tiling_on_sc=False),
    out_type=[
        jax.ShapeDtypeStruct((NC, NPAD, D), jnp.float32),
        jax.ShapeDtypeStruct((NC, NPAD, 16), jnp.float32),
    ],
    scratch_types=[
        pltpu.VMEM((BATCH,), jnp.int32),
        pltpu.VMEM((BATCH,), jnp.int32),
        pltpu.VMEM((BATCH,), jnp.int32),
        pltpu.VMEM((BATCH,), jnp.int32),
        pltpu.VMEM((BATCH, D), jnp.float32),
        pltpu.VMEM((BATCH, D), jnp.float32),
        pltpu.VMEM((BATCH, 16), jnp.float32),
        pltpu.VMEM((CH, D), jnp.float32),
        pltpu.VMEM((CH, 16), jnp.float32),
        pltpu.VMEM_SHARED((NPAD, D), jnp.float32),
        pltpu.VMEM_SHARED((NPAD, 16), jnp.float32),
        pltpu.SemaphoreType.DMA,
        pltpu.SemaphoreType.DMA,
        pltpu.SemaphoreType.DMA,
        pltpu.SemaphoreType.DMA,
        pltpu.SemaphoreType.DMA,
        pltpu.SemaphoreType.DMA,
        pltpu.SemaphoreType.DMA,
        pltpu.SemaphoreType.DMA,
        pltpu.SemaphoreType.DMA,
        pltpu.SemaphoreType.DMA,
    ],
)
def _sc_agg(t_hbm, src_hbm, dst_hbm, zrow_hbm, zcnt_hbm, ones_hbm,
            pacc_hbm, pcnt_hbm,
            src_v0, src_v1, dst_v0, dst_v1, rows_v0, rows_v1,
            ones_v, stage_v, stagec_v, acc_s, cnt_s,
            gsem0, gsem1, rsem0, rsem1, csem0, csem1,
            issem0, issem1, idsem0, idsem1):
    c = lax.axis_index("c")
    s = lax.axis_index("s")
    wid = s * NC + c

    SRC = (src_v0, src_v1)
    DST = (dst_v0, dst_v1)
    ROWS = (rows_v0, rows_v1)
    GS = (gsem0, gsem1)
    RS = (rsem0, rsem1)
    CS = (csem0, csem1)
    ISS = (issem0, issem1)
    IDS = (idsem0, idsem1)

    def off(i):
        return pl.multiple_of(wid * EPW + i * BATCH, 8)

    def idx_src(i, p):
        return pltpu.make_async_copy(src_hbm.at[pl.ds(off(i), BATCH)], SRC[p], ISS[p])

    def idx_dst(i, p):
        return pltpu.make_async_copy(dst_hbm.at[pl.ds(off(i), BATCH)], DST[p], IDS[p])

    def gath(p):
        return pltpu.make_async_copy(t_hbm.at[SRC[p]], ROWS[p], GS[p])

    def srow(p):
        return pltpu.make_async_copy(ROWS[p], acc_s.at[DST[p]], RS[p])

    def scnt(p):
        return pltpu.make_async_copy(ones_v, cnt_s.at[DST[p]], CS[p])

    pltpu.sync_copy(zrow_hbm, stage_v)
    pltpu.sync_copy(zcnt_hbm, stagec_v)
    pltpu.sync_copy(ones_hbm, ones_v)

    def zbody(j, carry):
        o = pl.multiple_of(s * RPT + j * CH, 8)
        pltpu.sync_copy(stage_v, acc_s.at[pl.ds(o, CH)])
        pltpu.sync_copy(stagec_v, cnt_s.at[pl.ds(o, CH)])
        return carry

    lax.fori_loop(0, NCH, zbody, 0)
    plsc.subcore_barrier()

    idx_src(0, 0).start()
    idx_dst(0, 0).start()
    idx_src(1, 1).start()
    idx_src(0, 0).wait()
    gath(0).start()

    gath(0).wait()
    idx_dst(0, 0).wait()
    srow(0).start(add=True)
    scnt(0).start(add=True)
    idx_src(1, 1).wait()
    gath(1).start()
    idx_dst(1, 1).start()
    idx_src(2, 0).start()

    gath(1).wait()
    idx_dst(1, 1).wait()
    srow(1).start(add=True)
    scnt(1).start(add=True)
    srow(0).wait()
    scnt(0).wait()
    idx_src(2, 0).wait()
    gath(0).start()
    idx_dst(2, 0).start()
    idx_src(3, 1).start()

    def step(i, p):
        q = 1 - p
        gath(p).wait()
        idx_dst(i, p).wait()
        srow(p).start(add=True)
        scnt(p).start(add=True)
        srow(q).wait()
        scnt(q).wait()
        idx_src(i + 1, q).wait()
        gath(q).start()
        idx_dst(i + 1, q).start()
        idx_src(i + 2, p).start()

    def pair(k, carry):
        step(2 * k, 0)
        step(2 * k + 1, 1)
        return carry

    lax.fori_loop(1, 61, pair, 0)
    step(122, 0)

    gath(1).wait()
    idx_dst(123, 1).wait()
    srow(1).start(add=True)
    scnt(1).start(add=True)
    srow(0).wait()
    scnt(0).wait()
    idx_src(124, 0).wait()
    gath(0).start()
    idx_dst(124, 0).start()

    gath(0).wait()
    idx_dst(124, 0).wait()
    srow(0).start(add=True)
    scnt(0).start(add=True)
    srow(1).wait()
    scnt(1).wait()

    srow(0).wait()
    scnt(0).wait()
    plsc.subcore_barrier()

    def wbody(j, carry):
        off = pl.multiple_of(s * RPT + j * CH, 8)
        pltpu.sync_copy(acc_s.at[pl.ds(off, CH)], stage_v)
        pltpu.sync_copy(stage_v, pacc_hbm.at[c, pl.ds(off, CH)])
        pltpu.sync_copy(cnt_s.at[pl.ds(off, CH)], stagec_v)
        pltpu.sync_copy(stagec_v, pcnt_hbm.at[c, pl.ds(off, CH)])
        return carry

    lax.fori_loop(0, NCH, wbody, 0)


_BLK = 1000
_GRID = N // _BLK


def _dot_t(a, w):
    return lax.dot_general(a, w, (((1,), (1,)), ((), ())),
                           preferred_element_type=jnp.float32)


def _tc_in_body(x_ref, wl_ref, wr_ref, b_ref, t_ref, r_ref):
    xb = x_ref[...]
    t_ref[...] = _dot_t(xb, wl_ref[...])
    r_ref[...] = _dot_t(xb, wr_ref[...]) + b_ref[...]


def _tc_in(x, W_l, W_r, b):
    w_spec = pl.BlockSpec((D, D), lambda i: (0, 0))
    return pl.pallas_call(
        _tc_in_body,
        grid=(_GRID,),
        in_specs=[pl.BlockSpec((_BLK, D), lambda i: (i, 0)), w_spec, w_spec,
                  pl.BlockSpec((1, D), lambda i: (0, 0))],
        out_specs=[pl.BlockSpec((_BLK, D), lambda i: (i, 0))] * 2,
        out_shape=[jax.ShapeDtypeStruct((N, D), jnp.float32)] * 2,
    )(x, W_l, W_r, b.reshape(1, D))


def _tc_mid_body(pacc_ref, pcnt_ref, r_ref, wl_ref, wr_ref, b_ref,
                 t_ref, r2_ref):
    agg = pacc_ref[0] + pacc_ref[1]
    cnt = pcnt_ref[0, :, 0:1] + pcnt_ref[1, :, 0:1]
    mean = agg / jnp.maximum(cnt, 1.0)
    h = jnp.maximum(mean + r_ref[...], 0.0)
    t_ref[...] = _dot_t(h, wl_ref[...])
    r2_ref[...] = _dot_t(h, wr_ref[...]) + b_ref[...]


def _tc_mid(pacc, pcnt, r1, W_l, W_r, b):
    w_spec = pl.BlockSpec((D, D), lambda i: (0, 0))
    return pl.pallas_call(
        _tc_mid_body,
        grid=(_GRID,),
        in_specs=[pl.BlockSpec((NC, _BLK, D), lambda i: (0, i, 0)),
                  pl.BlockSpec((NC, _BLK, 16), lambda i: (0, i, 0)),
                  pl.BlockSpec((_BLK, D), lambda i: (i, 0)),
                  w_spec, w_spec,
                  pl.BlockSpec((1, D), lambda i: (0, 0))],
        out_specs=[pl.BlockSpec((_BLK, D), lambda i: (i, 0))] * 2,
        out_shape=[jax.ShapeDtypeStruct((N, D), jnp.float32)] * 2,
    )(pacc, pcnt, r1, W_l, W_r, b.reshape(1, D))


def _tc_out_body(pacc_ref, pcnt_ref, r_ref, o_ref):
    agg = pacc_ref[0] + pacc_ref[1]
    cnt = pcnt_ref[0, :, 0:1] + pcnt_ref[1, :, 0:1]
    o_ref[...] = agg / jnp.maximum(cnt, 1.0) + r_ref[...]


def _tc_out(pacc, pcnt, r2):
    return pl.pallas_call(
        _tc_out_body,
        grid=(_GRID,),
        in_specs=[pl.BlockSpec((NC, _BLK, D), lambda i: (0, i, 0)),
                  pl.BlockSpec((NC, _BLK, 16), lambda i: (0, i, 0)),
                  pl.BlockSpec((_BLK, D), lambda i: (i, 0))],
        out_specs=pl.BlockSpec((_BLK, D), lambda i: (i, 0)),
        out_shape=jax.ShapeDtypeStruct((N, D), jnp.float32),
    )(pacc, pcnt, r2)


def kernel(x, edge_index, W1_l, b1_l, W1_r, W2_l, b2_l, W2_r):
    src = edge_index[0].astype(jnp.int32)
    dst = edge_index[1].astype(jnp.int32)
    zrow = jnp.zeros((CH, D), jnp.float32)
    zcnt = jnp.zeros((CH, 16), jnp.float32)
    ones = jnp.ones((BATCH, 16), jnp.float32)

    t1, r1 = _tc_in(x, W1_l, W1_r, b1_l)
    pacc1, pcnt1 = _sc_agg(t1, src, dst, zrow, zcnt, ones)
    t2, r2 = _tc_mid(pacc1, pcnt1, r1, W2_l, W2_r, b2_l)
    pacc2, _ = _sc_agg(t2, src, dst, zrow, zcnt, ones)
    return _tc_out(pacc2, pcnt1, r2)
